# baseline (speedup 1.0000x reference)
import functools

import jax
import jax.numpy as jnp
from jax import lax
from jax.experimental import pallas as pl
from jax.experimental.pallas import tpu as pltpu
from jax.experimental.pallas import tpu_sc as plsc

N = 10000
E = 320000
D = 128
DH = D // 2
NEG_SLOPE = 0.2

NC = 2
NS = 16
NW = NC * NS
NP = 10240
EP = NW * 10240
C = 128
NCHA = EP // NW // C
NCHB = EP // NS // C
NPS = NP // NS

_mesh = plsc.VectorSubcoreMesh(core_axis_name="c", subcore_axis_name="s")
_sc_params = pltpu.CompilerParams(needs_layout_passes=False,
                                  use_tc_tiling_on_sc=False)


def _tc_prep_body(x_ref, w_ref, a_ref, whs_ref, s1_ref, s2_ref):
    wh = lax.dot_general(x_ref[...], w_ref[...],
                         (((1,), (1,)), ((), ())),
                         preferred_element_type=jnp.float32)
    whb = wh.astype(jnp.bfloat16)
    whs_ref[0:NP, :] = whb[:, 0:DH]
    whs_ref[NP:2 * NP, :] = whb[:, DH:D]
    s1_ref[...] = wh @ a_ref[0:D]
    s2_ref[...] = wh @ a_ref[D:2 * D]


def _tc_prep(x_p, W, a):
    return pl.pallas_call(
        _tc_prep_body,
        out_shape=[
            jax.ShapeDtypeStruct((2 * NP, DH), jnp.bfloat16),
            jax.ShapeDtypeStruct((NP,), jnp.float32),
            jax.ShapeDtypeStruct((NP,), jnp.float32),
        ],
    )(x_p, W, a)


@functools.partial(
    pl.kernel,
    out_type=[
        jax.ShapeDtypeStruct((NW, NCHA, C), jnp.bfloat16),
        jax.ShapeDtypeStruct((NW, NP), jnp.float32),
    ],
    mesh=_mesh,
    compiler_params=_sc_params,
    scratch_types=[
        pltpu.VMEM((NCHA, C), jnp.int32),
        pltpu.VMEM((NCHA, C), jnp.int32),
        pltpu.VMEM((NP,), jnp.float32),
        pltpu.VMEM((NP,), jnp.float32),
        pltpu.VMEM((NCHA, C), jnp.bfloat16),
        pltpu.VMEM((NP,), jnp.float32),
        pltpu.SemaphoreType.DMA,
    ],
)
def _sc_phase_a(src_hbm, tgt_hbm, s1_hbm, s2_hbm, p_hbm, esums_hbm,
                src_v, tgt_v, s1_v, s2_v, p_v, esum_v, isem):
    cid = lax.axis_index("c")
    sid = lax.axis_index("s")
    wid = sid * NC + cid

    pltpu.async_copy(src_hbm.at[wid], src_v, isem)
    pltpu.async_copy(tgt_hbm.at[wid], tgt_v, isem)
    pltpu.async_copy(s1_hbm, s1_v, isem)
    pltpu.async_copy(s2_hbm, s2_v, isem)

    zero16 = jnp.zeros((16,), jnp.float32)

    def _zero(i, carry):
        esum_v[pl.ds(i * 16, 16)] = zero16
        return carry

    lax.fori_loop(0, NP // 16, _zero, 0)

    pltpu.make_async_copy(src_hbm.at[wid], src_v, isem).wait()
    pltpu.make_async_copy(tgt_hbm.at[wid], tgt_v, isem).wait()
    pltpu.make_async_copy(s1_hbm, s1_v, isem).wait()
    pltpu.make_async_copy(s2_hbm, s2_v, isem).wait()

    lane = lax.iota(jnp.int32, 16)
    base = wid * (EP // NW)

    def _chunk(j, carry):
        for kp in range(C // 32):
            ps = []
            for u in range(2):
                k = 2 * kp + u
                sv = src_v[j, pl.ds(k * 16, 16)]
                tv = tgt_v[j, pl.ds(k * 16, 16)]
                e = (plsc.load_gather(s1_v, [sv])
                     + plsc.load_gather(s2_v, [tv]))
                e = jnp.where(e >= 0.0, e, e * NEG_SLOPE)
                p = jnp.exp(e)
                gidx = base + j * C + k * 16 + lane
                p = jnp.where(gidx < E, p, 0.0)
                plsc.addupdate_scatter(esum_v, [tv], p)
                ps.append(p)
            p_v[j, pl.ds(kp * 32, 32)] = plsc.pack(
                ps[0], ps[1], format=plsc.PackFormat.INTERLEAVED)
        return carry

    lax.fori_loop(0, NCHA, _chunk, 0)

    pltpu.sync_copy(p_v, p_hbm.at[wid])
    pltpu.sync_copy(esum_v, esums_hbm.at[wid])


def _tc_esum_body(e_ref, o_ref):
    o_ref[...] = 1.0 / (jnp.sum(e_ref[...], axis=0) + 1e-10)


def _tc_esum(esums):
    return pl.pallas_call(
        _tc_esum_body,
        out_shape=jax.ShapeDtypeStruct((NP,), jnp.float32),
    )(esums)


@functools.partial(
    pl.kernel,
    out_type=jax.ShapeDtypeStruct((NP, D), jnp.float32),
    mesh=_mesh,
    compiler_params=_sc_params,
    scratch_types=[
        pltpu.VMEM((NP,), jnp.float32),
        pltpu.VMEM((NCHB, C), jnp.int32),
        pltpu.VMEM((NCHB, C), jnp.int32),
        pltpu.VMEM((NCHB, C), jnp.bfloat16),
        pltpu.VMEM((C,), jnp.float32),
        pltpu.VMEM((C, DH), jnp.bfloat16),
        pltpu.VMEM((C, DH), jnp.bfloat16),
        pltpu.VMEM((C, DH), jnp.float32),
        pltpu.VMEM((C, DH), jnp.float32),
        pltpu.VMEM_SHARED((NP, DH), jnp.float32),
        pltpu.SemaphoreType.DMA,
        pltpu.SemaphoreType.DMA,
        pltpu.SemaphoreType.DMA,
        pltpu.SemaphoreType.DMA,
        pltpu.SemaphoreType.DMA,
    ],
)
def _sc_phase_b(src_hbm, tgt_hbm, p_hbm, esum_hbm, whs_hbm, out_hbm,
                esum_v, src_v, tgt_v, p_v, alpha_v, rows0_v, rows1_v,
                fbuf0_v, fbuf1_v, out_sh, gsem0, gsem1, ssem0, ssem1, isem):
    cid = lax.axis_index("c")
    sid = lax.axis_index("s")

    pltpu.sync_copy(src_hbm.at[cid * NS + sid], src_v)
    pltpu.async_copy(whs_hbm.at[src_v.at[0]], rows0_v, gsem0)
    pltpu.async_copy(tgt_hbm.at[sid], tgt_v, isem)
    pltpu.async_copy(p_hbm.at[sid], p_v, isem)
    pltpu.async_copy(esum_hbm, esum_v, isem)

    zero16 = jnp.zeros((16,), jnp.float32)

    def _zrow(r, carry):
        for q in range(DH // 16):
            fbuf0_v[r, pl.ds(q * 16, 16)] = zero16
        return carry

    lax.fori_loop(0, C, _zrow, 0)
    for m in range(NPS // C):
        pltpu.sync_copy(fbuf0_v, out_sh.at[pl.ds(sid * NPS + m * C, C)])

    pltpu.make_async_copy(tgt_hbm.at[sid], tgt_v, isem).wait()
    pltpu.make_async_copy(p_hbm.at[sid], p_v, isem).wait()
    pltpu.make_async_copy(esum_hbm, esum_v, isem).wait()

    plsc.subcore_barrier()

    bufs = (rows0_v, rows1_v)
    fbufs = (fbuf0_v, fbuf1_v)
    gsems = (gsem0, gsem1)
    ssems = (ssem0, ssem1)
    himask = jnp.full((16,), -65536, jnp.int32)

    def _body(i, carry):
        for b in range(2):
            j = 2 * i + b
            rows_b = bufs[b]
            fbuf_b = fbufs[b]
            for kp in range(C // 32):
                pb = p_v[j, pl.ds(kp * 32, 32)]
                pi = plsc.bitcast(pb, jnp.int32)
                pg = (plsc.bitcast(pi << 16, jnp.float32),
                      plsc.bitcast(pi & himask, jnp.float32))
                for u in range(2):
                    k = 2 * kp + u
                    tv = tgt_v[j, pl.ds(k * 16, 16)]
                    rin = plsc.load_gather(esum_v, [tv])
                    alpha_v[pl.ds(k * 16, 16)] = pg[u] * rin

            pltpu.make_async_copy(whs_hbm.at[src_v.at[j]], rows_b,
                                  gsems[b]).wait()

            @pl.when(j + 1 < NCHB)
            def _prefetch():
                pltpu.async_copy(whs_hbm.at[src_v.at[j + 1]], bufs[1 - b],
                                 gsems[1 - b])

            @pl.when(j >= 2)
            def _drain():
                pltpu.make_async_copy(fbuf_b, out_sh.at[tgt_v.at[j - 2]],
                                      ssems[b]).wait()

            def _scale(g, c2):
                for u in range(2):
                    al16 = alpha_v[pl.ds((2 * g + u) * 16, 16)]
                    for l in range(16):
                        av = jnp.full((16,), al16[l], jnp.float32)
                        r = (2 * g + u) * 16 + l
                        for h in range(DH // 32):
                            vb = rows_b[r, pl.ds(h * 32, 32)]
                            vi = plsc.bitcast(vb, jnp.int32)
                            lo = plsc.bitcast(vi << 16, jnp.float32)
                            hi = plsc.bitcast(vi & himask, jnp.float32)
                            fbuf_b[r, pl.ds(h * 32, 16)] = lo * av
                            fbuf_b[r, pl.ds(h * 32 + 16, 16)] = hi * av
                return c2

            lax.fori_loop(0, C // 32, _scale, 0)
            pltpu.async_copy(fbuf_b, out_sh.at[tgt_v.at[j]], ssems[b],
                             add=True)
        return carry

    lax.fori_loop(0, NCHB // 2, _body, 0)
    for b in range(2):
        pltpu.make_async_copy(fbufs[b], out_sh.at[tgt_v.at[NCHB - 2 + b]],
                              ssems[b]).wait()

    plsc.subcore_barrier()
    pltpu.sync_copy(out_sh.at[pl.ds(sid * NPS, NPS)],
                    out_hbm.at[pl.ds(sid * NPS, NPS),
                               pl.ds(cid * DH, DH)])


def _row_perm():
    p64 = []
    for h32 in range(2):
        p64 += [h32 * 32 + 2 * m for m in range(16)]
        p64 += [h32 * 32 + 2 * m + 1 for m in range(16)]
    i64 = [0] * DH
    for s, c in enumerate(p64):
        i64[c] = s
    return tuple(h * DH + i64[c] for h in range(2) for c in range(DH))


_RP = _row_perm()


def kernel(x, edge_index, W, a):
    rp = jnp.asarray(_RP, jnp.int32)
    W2 = jnp.take(W, rp, axis=0)
    a_p = jnp.concatenate([jnp.take(a[:D], rp), jnp.take(a[D:], rp)])
    x_p = jnp.pad(x, ((0, NP - N), (0, 0)))
    ei_p = jnp.pad(edge_index, ((0, 0), (0, EP - E)))
    src_f, tgt_f = ei_p[0], ei_p[1]
    src16 = src_f.reshape(NS, NCHB, C)
    src2 = jnp.concatenate([src16, src16 + NP])

    whs, s1, s2 = _tc_prep(x_p, W2, a_p)
    p, esums = _sc_phase_a(src_f.reshape(NW, NCHA, C),
                           tgt_f.reshape(NW, NCHA, C), s1, s2)
    esum = _tc_esum(esums)
    out = _sc_phase_b(src2, tgt_f.reshape(NS, NCHB, C),
                      p.reshape(NS, NCHB, C), esum, whs)
    return out[:N]


# speedup vs baseline: 1.0482x; 1.0482x over previous
import functools

import jax
import jax.numpy as jnp
from jax import lax
from jax.experimental import pallas as pl
from jax.experimental.pallas import tpu as pltpu
from jax.experimental.pallas import tpu_sc as plsc

N = 10000
E = 320000
D = 128
DH = D // 2
NEG_SLOPE = 0.2

NC = 2
NS = 16
NW = NC * NS
NP = 10240
EP = NW * 10240
C = 128
NCHA = EP // NW // C
NCHB = EP // NS // C
NPS = NP // NS

_mesh = plsc.VectorSubcoreMesh(core_axis_name="c", subcore_axis_name="s")
_sc_params = pltpu.CompilerParams(needs_layout_passes=False,
                                  use_tc_tiling_on_sc=False)


def _tc_prep_body(x_ref, w_ref, a_ref, whs_ref, s1_ref, s2_ref):
    wh = lax.dot_general(x_ref[...], w_ref[...],
                         (((1,), (1,)), ((), ())),
                         preferred_element_type=jnp.float32)
    whb = wh.astype(jnp.bfloat16)
    whs_ref[0:NP, :] = whb[:, 0:DH]
    whs_ref[NP:2 * NP, :] = whb[:, DH:D]
    s1_ref[...] = wh @ a_ref[0:D]
    s2_ref[...] = wh @ a_ref[D:2 * D]


def _tc_prep(x_p, W, a):
    return pl.pallas_call(
        _tc_prep_body,
        out_shape=[
            jax.ShapeDtypeStruct((2 * NP, DH), jnp.bfloat16),
            jax.ShapeDtypeStruct((NP,), jnp.float32),
            jax.ShapeDtypeStruct((NP,), jnp.float32),
        ],
    )(x_p, W, a)


@functools.partial(
    pl.kernel,
    out_type=[
        jax.ShapeDtypeStruct((NW, NCHA, C), jnp.bfloat16),
        jax.ShapeDtypeStruct((NW, NP), jnp.float32),
    ],
    mesh=_mesh,
    compiler_params=_sc_params,
    scratch_types=[
        pltpu.VMEM((NCHA, C), jnp.int32),
        pltpu.VMEM((NCHA, C), jnp.int32),
        pltpu.VMEM((NP,), jnp.float32),
        pltpu.VMEM((NP,), jnp.float32),
        pltpu.VMEM((NCHA, C), jnp.bfloat16),
        pltpu.VMEM((NP,), jnp.float32),
        pltpu.SemaphoreType.DMA,
    ],
)
def _sc_phase_a(src_hbm, tgt_hbm, s1_hbm, s2_hbm, p_hbm, esums_hbm,
                src_v, tgt_v, s1_v, s2_v, p_v, esum_v, isem):
    cid = lax.axis_index("c")
    sid = lax.axis_index("s")
    wid = sid * NC + cid

    pltpu.async_copy(src_hbm.at[wid], src_v, isem)
    pltpu.async_copy(tgt_hbm.at[wid], tgt_v, isem)
    pltpu.async_copy(s1_hbm, s1_v, isem)
    pltpu.async_copy(s2_hbm, s2_v, isem)

    zero16 = jnp.zeros((16,), jnp.float32)

    def _zero(i, carry):
        esum_v[pl.ds(i * 16, 16)] = zero16
        return carry

    lax.fori_loop(0, NP // 16, _zero, 0)

    pltpu.make_async_copy(src_hbm.at[wid], src_v, isem).wait()
    pltpu.make_async_copy(tgt_hbm.at[wid], tgt_v, isem).wait()
    pltpu.make_async_copy(s1_hbm, s1_v, isem).wait()
    pltpu.make_async_copy(s2_hbm, s2_v, isem).wait()

    lane = lax.iota(jnp.int32, 16)
    base = wid * (EP // NW)

    def _chunk(j, carry):
        for kp in range(C // 32):
            ps = []
            for u in range(2):
                k = 2 * kp + u
                sv = src_v[j, pl.ds(k * 16, 16)]
                tv = tgt_v[j, pl.ds(k * 16, 16)]
                e = (plsc.load_gather(s1_v, [sv])
                     + plsc.load_gather(s2_v, [tv]))
                e = jnp.where(e >= 0.0, e, e * NEG_SLOPE)
                p = jnp.exp(e)
                gidx = base + j * C + k * 16 + lane
                p = jnp.where(gidx < E, p, 0.0)
                plsc.addupdate_scatter(esum_v, [tv], p)
                ps.append(p)
            p_v[j, pl.ds(kp * 32, 32)] = plsc.pack(
                ps[0], ps[1], format=plsc.PackFormat.INTERLEAVED)
        return carry

    lax.fori_loop(0, NCHA, _chunk, 0)

    pltpu.sync_copy(p_v, p_hbm.at[wid])
    pltpu.sync_copy(esum_v, esums_hbm.at[wid])


def _tc_esum_body(e_ref, o_ref):
    o_ref[...] = 1.0 / (jnp.sum(e_ref[...], axis=0) + 1e-10)


def _tc_esum(esums):
    return pl.pallas_call(
        _tc_esum_body,
        out_shape=jax.ShapeDtypeStruct((NP,), jnp.float32),
    )(esums)


@functools.partial(
    pl.kernel,
    out_type=jax.ShapeDtypeStruct((NC, NP, DH), jnp.float32),
    mesh=_mesh,
    compiler_params=_sc_params,
    scratch_types=[
        pltpu.VMEM((NP,), jnp.float32),
        pltpu.VMEM((NCHB, C), jnp.int32),
        pltpu.VMEM((NCHB, C), jnp.int32),
        pltpu.VMEM((NCHB, C), jnp.bfloat16),
        pltpu.VMEM((C,), jnp.float32),
        pltpu.VMEM((C, DH), jnp.bfloat16),
        pltpu.VMEM((C, DH), jnp.bfloat16),
        pltpu.VMEM((C, DH), jnp.float32),
        pltpu.VMEM((C, DH), jnp.float32),
        pltpu.VMEM_SHARED((NP, DH), jnp.float32),
        pltpu.SemaphoreType.DMA,
        pltpu.SemaphoreType.DMA,
        pltpu.SemaphoreType.DMA,
        pltpu.SemaphoreType.DMA,
        pltpu.SemaphoreType.DMA,
    ],
)
def _sc_phase_b(src_hbm, tgt_hbm, p_hbm, esum_hbm, whs_hbm, out_hbm,
                esum_v, src_v, tgt_v, p_v, alpha_v, rows0_v, rows1_v,
                fbuf0_v, fbuf1_v, out_sh, gsem0, gsem1, ssem0, ssem1, isem):
    cid = lax.axis_index("c")
    sid = lax.axis_index("s")

    pltpu.sync_copy(src_hbm.at[cid * NS + sid], src_v)
    pltpu.async_copy(whs_hbm.at[src_v.at[0]], rows0_v, gsem0)
    pltpu.async_copy(tgt_hbm.at[sid], tgt_v, isem)
    pltpu.async_copy(p_hbm.at[sid], p_v, isem)
    pltpu.async_copy(esum_hbm, esum_v, isem)

    zero16 = jnp.zeros((16,), jnp.float32)

    def _zrow(r, carry):
        for q in range(DH // 16):
            fbuf0_v[r, pl.ds(q * 16, 16)] = zero16
        return carry

    lax.fori_loop(0, C, _zrow, 0)
    for m in range(NPS // C):
        pltpu.sync_copy(fbuf0_v, out_sh.at[pl.ds(sid * NPS + m * C, C)])

    pltpu.make_async_copy(tgt_hbm.at[sid], tgt_v, isem).wait()
    pltpu.make_async_copy(p_hbm.at[sid], p_v, isem).wait()
    pltpu.make_async_copy(esum_hbm, esum_v, isem).wait()

    plsc.subcore_barrier()

    bufs = (rows0_v, rows1_v)
    fbufs = (fbuf0_v, fbuf1_v)
    gsems = (gsem0, gsem1)
    ssems = (ssem0, ssem1)
    himask = jnp.full((16,), -65536, jnp.int32)

    def _body(i, carry):
        for b in range(2):
            j = 2 * i + b
            rows_b = bufs[b]
            fbuf_b = fbufs[b]
            for kp in range(C // 32):
                pb = p_v[j, pl.ds(kp * 32, 32)]
                pi = plsc.bitcast(pb, jnp.int32)
                pg = (plsc.bitcast(pi << 16, jnp.float32),
                      plsc.bitcast(pi & himask, jnp.float32))
                for u in range(2):
                    k = 2 * kp + u
                    tv = tgt_v[j, pl.ds(k * 16, 16)]
                    rin = plsc.load_gather(esum_v, [tv])
                    alpha_v[pl.ds(k * 16, 16)] = pg[u] * rin

            pltpu.make_async_copy(whs_hbm.at[src_v.at[j]], rows_b,
                                  gsems[b]).wait()

            @pl.when(j + 1 < NCHB)
            def _prefetch():
                pltpu.async_copy(whs_hbm.at[src_v.at[j + 1]], bufs[1 - b],
                                 gsems[1 - b])

            @pl.when(j >= 2)
            def _drain():
                pltpu.make_async_copy(fbuf_b, out_sh.at[tgt_v.at[j - 2]],
                                      ssems[b]).wait()

            def _scale(g, c2):
                for u in range(2):
                    al16 = alpha_v[pl.ds((2 * g + u) * 16, 16)]
                    for l in range(16):
                        av = jnp.full((16,), al16[l], jnp.float32)
                        r = (2 * g + u) * 16 + l
                        for h in range(DH // 32):
                            vb = rows_b[r, pl.ds(h * 32, 32)]
                            vi = plsc.bitcast(vb, jnp.int32)
                            lo = plsc.bitcast(vi << 16, jnp.float32)
                            hi = plsc.bitcast(vi & himask, jnp.float32)
                            fbuf_b[r, pl.ds(h * 32, 16)] = lo * av
                            fbuf_b[r, pl.ds(h * 32 + 16, 16)] = hi * av
                return c2

            lax.fori_loop(0, C // 32, _scale, 0)
            pltpu.async_copy(fbuf_b, out_sh.at[tgt_v.at[j]], ssems[b],
                             add=True)
        return carry

    lax.fori_loop(0, NCHB // 2, _body, 0)
    for b in range(2):
        pltpu.make_async_copy(fbufs[b], out_sh.at[tgt_v.at[NCHB - 2 + b]],
                              ssems[b]).wait()

    plsc.subcore_barrier()
    pltpu.sync_copy(out_sh.at[pl.ds(sid * NPS, NPS)],
                    out_hbm.at[cid, pl.ds(sid * NPS, NPS)])


def _row_perm():
    p64 = []
    for h32 in range(2):
        p64 += [h32 * 32 + 2 * m for m in range(16)]
        p64 += [h32 * 32 + 2 * m + 1 for m in range(16)]
    i64 = [0] * DH
    for s, c in enumerate(p64):
        i64[c] = s
    return tuple(h * DH + i64[c] for h in range(2) for c in range(DH))


_RP = _row_perm()


def kernel(x, edge_index, W, a):
    rp = jnp.asarray(_RP, jnp.int32)
    W2 = jnp.take(W, rp, axis=0)
    a_p = jnp.concatenate([jnp.take(a[:D], rp), jnp.take(a[D:], rp)])
    x_p = jnp.pad(x, ((0, NP - N), (0, 0)))
    ei_p = jnp.pad(edge_index, ((0, 0), (0, EP - E)))
    src_f, tgt_f = ei_p[0], ei_p[1]
    src16 = src_f.reshape(NS, NCHB, C)
    src2 = jnp.concatenate([src16, src16 + NP])

    whs, s1, s2 = _tc_prep(x_p, W2, a_p)
    p, esums = _sc_phase_a(src_f.reshape(NW, NCHA, C),
                           tgt_f.reshape(NW, NCHA, C), s1, s2)
    esum = _tc_esum(esums)
    parts = _sc_phase_b(src2, tgt_f.reshape(NS, NCHB, C),
                        p.reshape(NS, NCHB, C), esum, whs)
    return jnp.concatenate([parts[0], parts[1]], axis=1)[:N]
